# baseline (speedup 1.0000x reference)
import functools
import jax
import jax.numpy as jnp
from jax import lax
from jax.experimental import pallas as pl
from jax.experimental.pallas import tpu as pltpu
from jax.experimental.pallas import tpu_sc as plsc

_ORDER = 10
_N = 10000
_E = 320000
_RANK = 128

_NSUB = 16
_NCORE = 2
_HALF = _RANK // _NCORE
_CH = 128
_EPT = -(-_E // _NSUB)
_NCHUNK = -(-_EPT // _CH)
_EPAD = _NSUB * _NCHUNK * _CH
_NACC = ((_N + _NSUB + _NSUB - 1) // _NSUB) * _NSUB
_RPT = _N // _NSUB
_RSC = 125
_NB = 4
_NI = 7


def _body(xin, edg, alph, out, xb, acc, xsp, rows, idxb, sbuf, alpha_v,
          isem, gsem, hsem, ssem):
    c = lax.axis_index("c")
    s = lax.axis_index("s")
    f0 = c * _HALF

    zvec = jnp.zeros((16,), jnp.float32)

    def zero_rows0():
        def zrow(r, carry):
            for fg in range(_HALF // 16):
                rows[0, r, pl.ds(fg * 16, 16)] = zvec
            return carry

        lax.fori_loop(0, _CH, zrow, 0)

    zero_rows0()

    def scale_rows(ref):
        a = [alpha_v[pl.ds(fg * 16, 16)] for fg in range(_HALF // 16)]

        def srow(r, carry):
            for fg in range(_HALF // 16):
                sl = pl.ds(fg * 16, 16)
                ref[r, sl] = ref[r, sl] * a[fg]
            return carry

        lax.fori_loop(0, _RSC, srow, 0)

    zb = s * _RPT
    zsrc = rows.at[0].at[pl.ds(0, _RSC)]
    for k in range(_RPT // _RSC):
        pltpu.sync_copy(zsrc, acc.at[pl.ds(zb + k * _RSC, _RSC)])
    pltpu.sync_copy(rows.at[0].at[pl.ds(0, 1)], acc.at[pl.ds(_N + s, 1)])

    pltpu.sync_copy(alph.at[0, pl.ds(f0, _HALF)], alpha_v)
    for k in range(_RPT // _RSC):
        r0 = s * _RPT + k * _RSC
        pltpu.sync_copy(xin.at[pl.ds(r0, _RSC), pl.ds(f0, _HALF)], sbuf)
        scale_rows(sbuf)
        pltpu.sync_copy(sbuf, out.at[pl.ds(r0, _RSC), 0, pl.ds(f0, _HALF)])
        pltpu.sync_copy(sbuf, xsp.at[pl.ds(r0, _RSC)])
        pltpu.sync_copy(sbuf, xb.at[c, pl.ds(r0, _RSC)])
    plsc.subcore_barrier()

    def idx_cp(j, bi):
        return pltpu.make_async_copy(edg.at[s, j], idxb.at[bi], isem)

    def gather_cp_s(b, bi):
        return pltpu.make_async_copy(
            xsp.at[idxb.at[bi, 0]], rows.at[b], gsem)

    def gather_cp_h(b, bi):
        return pltpu.make_async_copy(
            xb.at[c].at[idxb.at[bi, 0]], rows.at[b], hsem)

    def gather_start(j, b, bi):
        on_sp = lax.rem(j, 3) == 2

        @pl.when(on_sp)
        def _():
            gather_cp_s(b, bi).start()

        @pl.when(jnp.logical_not(on_sp))
        def _():
            gather_cp_h(b, bi).start()

    def gather_wait(j, b, bi):
        on_sp = lax.rem(j, 3) == 2

        @pl.when(on_sp)
        def _():
            gather_cp_s(b, bi).wait()

        @pl.when(jnp.logical_not(on_sp))
        def _():
            gather_cp_h(b, bi).wait()

    def scatter_cp(j, b, bi):
        del j
        return pltpu.make_async_copy(
            rows.at[b], acc.at[idxb.at[bi, 1]], ssem)

    def iteration(i, carry):
        for j0 in range(5):
            idx_cp(j0, j0).start()
        for j0 in range(2):
            idx_cp(j0, j0).wait()
            gather_start(j0, j0, j0)

        def chunk(j, carry2):
            @pl.when(j >= 2)
            def _():
                scatter_cp(j - 2, lax.rem(j - 2, _NB),
                           lax.rem(j - 2, _NI)).wait()

            @pl.when(j + 2 < _NCHUNK)
            def _():
                idx_cp(j + 2, lax.rem(j + 2, _NI)).wait()
                gather_start(j + 2, lax.rem(j + 2, _NB),
                             lax.rem(j + 2, _NI))

            @pl.when(j + 5 < _NCHUNK)
            def _():
                idx_cp(j + 5, lax.rem(j + 5, _NI)).start()

            b = lax.rem(j, _NB)
            bi = lax.rem(j, _NI)
            gather_wait(j, b, bi)
            pltpu.async_copy(rows.at[b], acc.at[idxb.at[bi, 1]], ssem,
                             add=True)
            return carry2

        lax.fori_loop(0, _NCHUNK, chunk, 0)
        for jl in range(_NCHUNK - 2, _NCHUNK):
            scatter_cp(jl, jl % _NB, jl % _NI).wait()
        plsc.subcore_barrier()

        zero_rows0()
        pltpu.sync_copy(alph.at[i, pl.ds(f0, _HALF)], alpha_v)
        for k in range(_RPT // _RSC):
            r0 = s * _RPT + k * _RSC
            pltpu.sync_copy(acc.at[pl.ds(r0, _RSC)], sbuf)
            pltpu.sync_copy(rows.at[0].at[pl.ds(0, _RSC)],
                            acc.at[pl.ds(r0, _RSC)])
            scale_rows(sbuf)
            pltpu.sync_copy(sbuf, out.at[pl.ds(r0, _RSC), i, pl.ds(f0, _HALF)])
            pltpu.sync_copy(sbuf, xsp.at[pl.ds(r0, _RSC)])
            pltpu.sync_copy(sbuf, xb.at[c, pl.ds(r0, _RSC)])
        pltpu.sync_copy(rows.at[0].at[pl.ds(0, 1)], acc.at[pl.ds(_N + s, 1)])
        plsc.subcore_barrier()
        return carry

    lax.fori_loop(1, _ORDER + 1, iteration, 0)


@jax.jit
def _poly_conv(xin, edg, alph):
    mesh = plsc.VectorSubcoreMesh(core_axis_name="c", subcore_axis_name="s")
    f = pl.kernel(
        _body,
        out_type=(
            jax.ShapeDtypeStruct((_N, _ORDER + 1, _RANK), jnp.float32),
            jax.ShapeDtypeStruct((_NCORE, _N, _HALF), jnp.float32),
        ),
        mesh=mesh,
        scratch_types=[
            pltpu.VMEM_SHARED((_NACC, _HALF), jnp.float32),
            pltpu.VMEM_SHARED((_N, _HALF), jnp.float32),
            pltpu.VMEM((_NB, _CH, _HALF), jnp.float32),
            pltpu.VMEM((_NI, 2, _CH), jnp.int32),
            pltpu.VMEM((_RSC, _HALF), jnp.float32),
            pltpu.VMEM((_HALF,), jnp.float32),
            pltpu.SemaphoreType.DMA,
            pltpu.SemaphoreType.DMA,
            pltpu.SemaphoreType.DMA,
            pltpu.SemaphoreType.DMA,
        ],
        compiler_params=pltpu.CompilerParams(use_tc_tiling_on_sc=False),
    )
    return f(xin, edg, alph)[0]


def kernel(inputs, edge_index, weight):
    alphas = weight * jnp.tanh(1.0 / (weight + 1e-05))
    alph = alphas.reshape(_ORDER + 1, _RANK)

    src = edge_index[0]
    dst = edge_index[1]
    pad = _EPAD - _E
    srcp = jnp.concatenate(
        [src, jnp.zeros((pad,), jnp.int32)]).reshape(_NSUB, _NCHUNK, _CH)
    dstp = jnp.concatenate(
        [dst, _N + (jnp.arange(pad, dtype=jnp.int32) % _NSUB)]
    ).reshape(_NSUB, _NCHUNK, _CH)
    edg = jnp.stack([srcp, dstp], axis=2)

    return _poly_conv(inputs, edg, alph)


# speedup vs baseline: 1.1106x; 1.1106x over previous
import functools
import jax
import jax.numpy as jnp
from jax import lax
from jax.experimental import pallas as pl
from jax.experimental.pallas import tpu as pltpu
from jax.experimental.pallas import tpu_sc as plsc

_ORDER = 10
_N = 10000
_E = 320000
_RANK = 128

_NSUB = 16
_NCORE = 2
_HALF = _RANK // _NCORE
_CH = 128
_EPT = -(-_E // _NSUB)
_NCHUNK = -(-_EPT // _CH)
_EPAD = _NSUB * _NCHUNK * _CH
_NACC = ((_N + _NSUB + _NSUB - 1) // _NSUB) * _NSUB
_RPT = _N // _NSUB
_RSC = 125
_NB = 4
_NI = 7


def _body(xin, edg, alph, out, xb, acc, xsp, rows, idxb, sbuf, alpha_v,
          isem, gsem, hsem, ssem):
    c = lax.axis_index("c")
    s = lax.axis_index("s")
    f0 = c * _HALF

    zvec = jnp.zeros((16,), jnp.float32)

    def zero_rows0():
        def zrow(r, carry):
            for fg in range(_HALF // 16):
                rows[0, r, pl.ds(fg * 16, 16)] = zvec
            return carry

        lax.fori_loop(0, _CH, zrow, 0)

    zero_rows0()

    def scale_rows(ref):
        a = [alpha_v[pl.ds(fg * 16, 16)] for fg in range(_HALF // 16)]

        def srow(r, carry):
            for fg in range(_HALF // 16):
                sl = pl.ds(fg * 16, 16)
                ref[r, sl] = ref[r, sl] * a[fg]
            return carry

        lax.fori_loop(0, _RSC, srow, 0)

    zb = s * _RPT
    zsrc = rows.at[0].at[pl.ds(0, _RSC)]
    for k in range(_RPT // _RSC):
        pltpu.sync_copy(zsrc, acc.at[pl.ds(zb + k * _RSC, _RSC)])
    pltpu.sync_copy(rows.at[0].at[pl.ds(0, 1)], acc.at[pl.ds(_N + s, 1)])

    pltpu.sync_copy(alph.at[0, pl.ds(f0, _HALF)], alpha_v)
    for k in range(_RPT // _RSC):
        r0 = s * _RPT + k * _RSC
        pltpu.sync_copy(xin.at[pl.ds(r0, _RSC), pl.ds(f0, _HALF)], sbuf)
        scale_rows(sbuf)
        pltpu.sync_copy(sbuf, out.at[pl.ds(r0, _RSC), 0, pl.ds(f0, _HALF)])
        pltpu.sync_copy(sbuf, xsp.at[pl.ds(r0, _RSC)])
        pltpu.sync_copy(sbuf, xb.at[c, pl.ds(r0, _RSC)])
    plsc.subcore_barrier()

    def idx_cp(j, bi):
        return pltpu.make_async_copy(edg.at[s, j], idxb.at[bi], isem)

    def gather_cp_s(b, bi):
        return pltpu.make_async_copy(
            xsp.at[idxb.at[bi, 0]], rows.at[b], gsem)

    def gather_cp_h(b, bi):
        return pltpu.make_async_copy(
            xb.at[c].at[idxb.at[bi, 0]], rows.at[b], hsem)

    def edge_loop(gcp):
        for j0 in range(5):
            idx_cp(j0, j0).start()
        for j0 in range(2):
            idx_cp(j0, j0).wait()
            gcp(j0, j0).start()

        def chunk(j, carry2):
            @pl.when(j >= 2)
            def _():
                scatter_cp(j - 2, lax.rem(j - 2, _NB),
                           lax.rem(j - 2, _NI)).wait()

            @pl.when(j + 2 < _NCHUNK)
            def _():
                idx_cp(j + 2, lax.rem(j + 2, _NI)).wait()
                gcp(lax.rem(j + 2, _NB), lax.rem(j + 2, _NI)).start()

            @pl.when(j + 5 < _NCHUNK)
            def _():
                idx_cp(j + 5, lax.rem(j + 5, _NI)).start()

            b = lax.rem(j, _NB)
            bi = lax.rem(j, _NI)
            gcp(b, bi).wait()
            pltpu.async_copy(rows.at[b], acc.at[idxb.at[bi, 1]], ssem,
                             add=True)
            return carry2

        lax.fori_loop(0, _NCHUNK, chunk, 0)
        for jl in range(_NCHUNK - 2, _NCHUNK):
            scatter_cp(jl, jl % _NB, jl % _NI).wait()

    def scatter_cp(j, b, bi):
        del j
        return pltpu.make_async_copy(
            rows.at[b], acc.at[idxb.at[bi, 1]], ssem)

    def iteration(i, carry):
        @pl.when(s < 11)
        def _():
            edge_loop(gather_cp_h)

        @pl.when(s >= 11)
        def _():
            edge_loop(gather_cp_s)

        plsc.subcore_barrier()

        zero_rows0()
        pltpu.sync_copy(alph.at[i, pl.ds(f0, _HALF)], alpha_v)
        for k in range(_RPT // _RSC):
            r0 = s * _RPT + k * _RSC
            pltpu.sync_copy(acc.at[pl.ds(r0, _RSC)], sbuf)
            pltpu.sync_copy(rows.at[0].at[pl.ds(0, _RSC)],
                            acc.at[pl.ds(r0, _RSC)])
            scale_rows(sbuf)
            pltpu.sync_copy(sbuf, out.at[pl.ds(r0, _RSC), i, pl.ds(f0, _HALF)])
            pltpu.sync_copy(sbuf, xsp.at[pl.ds(r0, _RSC)])
            pltpu.sync_copy(sbuf, xb.at[c, pl.ds(r0, _RSC)])
        pltpu.sync_copy(rows.at[0].at[pl.ds(0, 1)], acc.at[pl.ds(_N + s, 1)])
        plsc.subcore_barrier()
        return carry

    lax.fori_loop(1, _ORDER + 1, iteration, 0)


@jax.jit
def _poly_conv(xin, edg, alph):
    mesh = plsc.VectorSubcoreMesh(core_axis_name="c", subcore_axis_name="s")
    f = pl.kernel(
        _body,
        out_type=(
            jax.ShapeDtypeStruct((_N, _ORDER + 1, _RANK), jnp.float32),
            jax.ShapeDtypeStruct((_NCORE, _N, _HALF), jnp.float32),
        ),
        mesh=mesh,
        scratch_types=[
            pltpu.VMEM_SHARED((_NACC, _HALF), jnp.float32),
            pltpu.VMEM_SHARED((_N, _HALF), jnp.float32),
            pltpu.VMEM((_NB, _CH, _HALF), jnp.float32),
            pltpu.VMEM((_NI, 2, _CH), jnp.int32),
            pltpu.VMEM((_RSC, _HALF), jnp.float32),
            pltpu.VMEM((_HALF,), jnp.float32),
            pltpu.SemaphoreType.DMA,
            pltpu.SemaphoreType.DMA,
            pltpu.SemaphoreType.DMA,
            pltpu.SemaphoreType.DMA,
        ],
        compiler_params=pltpu.CompilerParams(use_tc_tiling_on_sc=False),
    )
    return f(xin, edg, alph)[0]


def kernel(inputs, edge_index, weight):
    alphas = weight * jnp.tanh(1.0 / (weight + 1e-05))
    alph = alphas.reshape(_ORDER + 1, _RANK)

    src = edge_index[0]
    dst = edge_index[1]
    pad = _EPAD - _E
    srcp = jnp.concatenate(
        [src, jnp.zeros((pad,), jnp.int32)]).reshape(_NSUB, _NCHUNK, _CH)
    dstp = jnp.concatenate(
        [dst, _N + (jnp.arange(pad, dtype=jnp.int32) % _NSUB)]
    ).reshape(_NSUB, _NCHUNK, _CH)
    edg = jnp.stack([srcp, dstp], axis=2)

    return _poly_conv(inputs, edg, alph)


# speedup vs baseline: 1.3622x; 1.2265x over previous
import functools
import jax
import jax.numpy as jnp
from jax import lax
from jax.experimental import pallas as pl
from jax.experimental.pallas import tpu as pltpu
from jax.experimental.pallas import tpu_sc as plsc

_ORDER = 10
_N = 10000
_E = 320000
_RANK = 128

_NSUB = 16
_NCORE = 2
_HALF = _RANK // _NCORE
_CH = 128
_EPT = -(-_E // _NSUB)
_NCHUNK = -(-_EPT // _CH)
_EPAD = _NSUB * _NCHUNK * _CH
_NACC = ((_N + _NSUB + _NSUB - 1) // _NSUB) * _NSUB
_RPT = _N // _NSUB
_RSC = 125
_NSC = _RPT // _RSC
_NB = 4
_NI = 7


def _body(xin, edg, alph, out, acc, xsp, rows, idxb, alpha_all,
          isem, gsem, whbm, wsp, ssem):
    c = lax.axis_index("c")
    s = lax.axis_index("s")
    f0 = c * _HALF

    pltpu.sync_copy(alph.at[:, pl.ds(f0, _HALF)], alpha_all)

    zvec = jnp.zeros((16,), jnp.float32)

    def zero_rows0():
        def zrow(r, carry):
            for fg in range(_HALF // 16):
                rows[0, r, pl.ds(fg * 16, 16)] = zvec
            return carry

        lax.fori_loop(0, _CH, zrow, 0)

    zero_rows0()

    def scale_emit(i, read_cp):
        a = [alpha_all[i, pl.ds(fg * 16, 16)] for fg in range(_HALF // 16)]

        def bref(k):
            return rows.at[1 + (k % 2)]

        def row0(k):
            return s * _RPT + k * _RSC

        def writes(k):
            src = bref(k).at[pl.ds(0, _RSC)]
            r0 = row0(k)
            return (
                pltpu.make_async_copy(
                    src, out.at[pl.ds(r0, _RSC), i, pl.ds(f0, _HALF)], whbm),
                pltpu.make_async_copy(src, xsp.at[pl.ds(r0, _RSC)], wsp),
                pltpu.make_async_copy(
                    rows.at[0].at[pl.ds(0, _RSC)], acc.at[pl.ds(r0, _RSC)],
                    wsp),
            )

        read_cp(0, bref(0).at[pl.ds(0, _RSC)]).start()
        for k in range(_NSC):
            read_cp(k, bref(k).at[pl.ds(0, _RSC)]).wait()
            if k >= 1:
                for w in writes(k - 1):
                    w.wait()
            if k + 1 < _NSC:
                read_cp(k + 1, bref(k + 1).at[pl.ds(0, _RSC)]).start()

            ref = bref(k)

            def srow(r, carry):
                for fg in range(_HALF // 16):
                    sl = pl.ds(fg * 16, 16)
                    ref[r, sl] = ref[r, sl] * a[fg]
                return carry

            lax.fori_loop(0, _RSC, srow, 0)
            for w in writes(k):
                w.start()
        pltpu.make_async_copy(
            rows.at[0].at[pl.ds(0, 1)], acc.at[pl.ds(_N + s, 1)], wsp).start()
        for w in writes(_NSC - 1):
            w.wait()
        pltpu.make_async_copy(
            rows.at[0].at[pl.ds(0, 1)], acc.at[pl.ds(_N + s, 1)], wsp).wait()

    def read_xin(k, dst):
        r0 = s * _RPT + k * _RSC
        return pltpu.make_async_copy(
            xin.at[pl.ds(r0, _RSC), pl.ds(f0, _HALF)], dst, isem)

    scale_emit(0, read_xin)
    plsc.subcore_barrier()

    def idx_cp(j, bi):
        return pltpu.make_async_copy(edg.at[s, j], idxb.at[bi], isem)

    def gather_cp(b, bi):
        return pltpu.make_async_copy(
            xsp.at[idxb.at[bi, 0]], rows.at[b], gsem)

    def scatter_cp(j, b, bi):
        del j
        return pltpu.make_async_copy(
            rows.at[b], acc.at[idxb.at[bi, 1]], ssem)

    def edge_loop():
        for j0 in range(5):
            idx_cp(j0, j0).start()
        for j0 in range(2):
            idx_cp(j0, j0).wait()
            gather_cp(j0, j0).start()

        def chunk(j, carry2):
            @pl.when(j >= 2)
            def _():
                scatter_cp(j - 2, lax.rem(j - 2, _NB),
                           lax.rem(j - 2, _NI)).wait()

            @pl.when(j + 2 < _NCHUNK)
            def _():
                idx_cp(j + 2, lax.rem(j + 2, _NI)).wait()
                gather_cp(lax.rem(j + 2, _NB), lax.rem(j + 2, _NI)).start()

            @pl.when(j + 5 < _NCHUNK)
            def _():
                idx_cp(j + 5, lax.rem(j + 5, _NI)).start()

            b = lax.rem(j, _NB)
            bi = lax.rem(j, _NI)
            gather_cp(b, bi).wait()
            pltpu.async_copy(rows.at[b], acc.at[idxb.at[bi, 1]], ssem,
                             add=True)
            return carry2

        lax.fori_loop(0, _NCHUNK, chunk, 0)
        for jl in range(_NCHUNK - 2, _NCHUNK):
            scatter_cp(jl, jl % _NB, jl % _NI).wait()

    def read_acc(k, dst):
        r0 = s * _RPT + k * _RSC
        return pltpu.make_async_copy(acc.at[pl.ds(r0, _RSC)], dst, gsem)

    def iteration(i, carry):
        edge_loop()
        plsc.subcore_barrier()
        zero_rows0()
        scale_emit(i, read_acc)
        plsc.subcore_barrier()
        return carry

    lax.fori_loop(1, _ORDER + 1, iteration, 0)


@jax.jit
def _poly_conv(xin, edg, alph):
    mesh = plsc.VectorSubcoreMesh(core_axis_name="c", subcore_axis_name="s")
    f = pl.kernel(
        _body,
        out_type=jax.ShapeDtypeStruct((_N, _ORDER + 1, _RANK), jnp.float32),
        mesh=mesh,
        scratch_types=[
            pltpu.VMEM_SHARED((_NACC, _HALF), jnp.float32),
            pltpu.VMEM_SHARED((_N, _HALF), jnp.float32),
            pltpu.VMEM((_NB, _CH, _HALF), jnp.float32),
            pltpu.VMEM((_NI, 2, _CH), jnp.int32),
            pltpu.VMEM((_ORDER + 1, _HALF), jnp.float32),
            pltpu.SemaphoreType.DMA,
            pltpu.SemaphoreType.DMA,
            pltpu.SemaphoreType.DMA,
            pltpu.SemaphoreType.DMA,
            pltpu.SemaphoreType.DMA,
        ],
        compiler_params=pltpu.CompilerParams(use_tc_tiling_on_sc=False),
    )
    return f(xin, edg, alph)


def kernel(inputs, edge_index, weight):
    alphas = weight * jnp.tanh(1.0 / (weight + 1e-05))
    alph = alphas.reshape(_ORDER + 1, _RANK)

    src = edge_index[0]
    dst = edge_index[1]
    pad = _EPAD - _E
    srcp = jnp.concatenate(
        [src, jnp.zeros((pad,), jnp.int32)]).reshape(_NSUB, _NCHUNK, _CH)
    dstp = jnp.concatenate(
        [dst, _N + (jnp.arange(pad, dtype=jnp.int32) % _NSUB)]
    ).reshape(_NSUB, _NCHUNK, _CH)
    edg = jnp.stack([srcp, dstp], axis=2)

    return _poly_conv(inputs, edg, alph)
